# Initial kernel scaffold; baseline (speedup 1.0000x reference)
import functools

import jax
import jax.numpy as jnp
from jax import lax
from jax.experimental import pallas as pl
from jax.experimental.pallas import tpu as pltpu
from jax.experimental.pallas import tpu_sc as plsc

N_NODES = 10000
D = 128
E = 320000
COEFS = (1.0, -0.5, 0.25, -0.125, 0.0625)

NC = 2
NS = 16
NW = NC * NS

CH = 128
NCH = 80
NPAIR = NCH // 2
EPW = NCH * CH
E_PAD = EPW * NW
N_PAD = 10240
ROWS_PER_SUB = N_PAD // NS
DEG_W = 8


def _sc_mesh():
    return plsc.VectorSubcoreMesh(core_axis_name="c", subcore_axis_name="s")


def _sc_deg(dstp, ones_hbm, zeros_hbm):
    @functools.partial(
        pl.kernel,
        out_type=jax.ShapeDtypeStruct((NC, N_PAD, DEG_W), jnp.float32),
        mesh=_sc_mesh(),
        scratch_types=[
            pltpu.VMEM((NCH, CH), jnp.int32),
            pltpu.VMEM((CH, DEG_W), jnp.float32),
            pltpu.VMEM_SHARED((N_PAD, DEG_W), jnp.float32),
        ],
    )
    def k(dstp_hbm, ones_h, zeros_h, out_hbm, dbuf, ones_v, dega):
        c = lax.axis_index("c")
        s = lax.axis_index("s")
        wid = c * NS + s
        pltpu.sync_copy(zeros_h, dega.at[pl.ds(s * ROWS_PER_SUB, ROWS_PER_SUB)])
        pltpu.sync_copy(dstp_hbm.at[wid], dbuf)
        pltpu.sync_copy(ones_h, ones_v)
        plsc.subcore_barrier()

        def body(j, carry):
            pltpu.sync_copy(ones_v, dega.at[dbuf.at[j]], add=True)
            return carry

        lax.fori_loop(0, NCH, body, 0)
        plsc.subcore_barrier()
        sl = pl.ds(s * ROWS_PER_SUB, ROWS_PER_SUB)
        pltpu.sync_copy(dega.at[sl], out_hbm.at[c].at[sl])

    return k(dstp, ones_hbm, zeros_hbm)


def _sc_round(g, srcp, dstp, zeros_hbm):
    @functools.partial(
        pl.kernel,
        out_type=jax.ShapeDtypeStruct((NC, N_PAD, D), jnp.float32),
        mesh=_sc_mesh(),
        scratch_types=[
            pltpu.VMEM((NCH, CH), jnp.int32),
            pltpu.VMEM((NCH, CH), jnp.int32),
            pltpu.VMEM((CH, D), jnp.float32),
            pltpu.VMEM((CH, D), jnp.float32),
            pltpu.VMEM_SHARED((N_PAD, D), jnp.float32),
            pltpu.SemaphoreType.DMA,
            pltpu.SemaphoreType.DMA,
        ],
    )
    def k(g_hbm, srcp_hbm, dstp_hbm, zeros_h, out_hbm,
          sbuf, dbuf, r0, r1, agg, sem0, sem1):
        c = lax.axis_index("c")
        s = lax.axis_index("s")
        wid = c * NS + s
        pltpu.sync_copy(zeros_h, agg.at[pl.ds(s * ROWS_PER_SUB, ROWS_PER_SUB)])
        pltpu.sync_copy(srcp_hbm.at[wid], sbuf)
        pltpu.sync_copy(dstp_hbm.at[wid], dbuf)
        plsc.subcore_barrier()

        pltpu.async_copy(g_hbm.at[sbuf.at[0]], r0, sem0)
        pltpu.async_copy(g_hbm.at[sbuf.at[1]], r1, sem1)

        def body(jj, carry):
            j0 = 2 * jj
            pltpu.make_async_copy(g_hbm.at[sbuf.at[j0]], r0, sem0).wait()
            pltpu.sync_copy(r0, agg.at[dbuf.at[j0]], add=True)
            pltpu.async_copy(g_hbm.at[sbuf.at[j0 + 2]], r0, sem0)
            pltpu.make_async_copy(g_hbm.at[sbuf.at[j0 + 1]], r1, sem1).wait()
            pltpu.sync_copy(r1, agg.at[dbuf.at[j0 + 1]], add=True)
            pltpu.async_copy(g_hbm.at[sbuf.at[j0 + 3]], r1, sem1)
            return carry

        lax.fori_loop(0, NPAIR - 1, body, 0)

        pltpu.make_async_copy(g_hbm.at[sbuf.at[NCH - 2]], r0, sem0).wait()
        pltpu.sync_copy(r0, agg.at[dbuf.at[NCH - 2]], add=True)
        pltpu.make_async_copy(g_hbm.at[sbuf.at[NCH - 1]], r1, sem1).wait()
        pltpu.sync_copy(r1, agg.at[dbuf.at[NCH - 1]], add=True)

        plsc.subcore_barrier()
        sl = pl.ds(s * ROWS_PER_SUB, ROWS_PER_SUB)
        pltpu.sync_copy(agg.at[sl], out_hbm.at[c].at[sl])

    return k(g, srcp, dstp, zeros_hbm)


_TC_BLK = 1280
_TC_GRID = N_PAD // _TC_BLK


def _row_spec(w):
    return pl.BlockSpec((_TC_BLK, w), lambda i: (i, 0))


def _tc_prep(d0, d1, x):
    def body(d0_ref, d1_ref, x_ref, dis_ref, g_ref):
        deg = d0_ref[:, 0:1] + d1_ref[:, 0:1]
        dis = lax.rsqrt(jnp.maximum(deg, 1.0))
        dis_ref[...] = dis
        g_ref[...] = x_ref[...] * dis

    return pl.pallas_call(
        body,
        grid=(_TC_GRID,),
        in_specs=[_row_spec(DEG_W), _row_spec(DEG_W), _row_spec(D)],
        out_specs=[_row_spec(1), _row_spec(D)],
        out_shape=[
            jax.ShapeDtypeStruct((N_PAD, 1), jnp.float32),
            jax.ShapeDtypeStruct((N_PAD, D), jnp.float32),
        ],
    )(d0, d1, x)


def _tc_dense(theta, p0, p1, feat, dis, h):
    def body(p0_ref, p1_ref, feat_ref, dis_ref, h_ref,
             feat_o, h_o, g_o):
        agg = p0_ref[...] + p1_ref[...]
        dis = dis_ref[...]
        f = feat_ref[...] - dis * agg
        feat_o[...] = f
        h_o[...] = h_ref[...] + theta * f
        g_o[...] = f * dis

    return pl.pallas_call(
        body,
        grid=(_TC_GRID,),
        in_specs=[_row_spec(D), _row_spec(D), _row_spec(D), _row_spec(1),
                  _row_spec(D)],
        out_specs=[_row_spec(D), _row_spec(D), _row_spec(D)],
        out_shape=[
            jax.ShapeDtypeStruct((N_PAD, D), jnp.float32),
            jax.ShapeDtypeStruct((N_PAD, D), jnp.float32),
            jax.ShapeDtypeStruct((N_PAD, D), jnp.float32),
        ],
    )(p0, p1, feat, dis, h)


def kernel(x, edge_index):
    src = edge_index[0].astype(jnp.int32)
    dst = edge_index[1].astype(jnp.int32)
    pad = E_PAD - E
    srcp = jnp.concatenate([src, jnp.zeros((pad,), jnp.int32)])
    dstp = jnp.concatenate([dst, jnp.full((pad,), N_NODES, jnp.int32)])
    srcp = srcp.reshape(NW, NCH, CH)
    dstp = dstp.reshape(NW, NCH, CH)

    x_pad = jnp.concatenate(
        [x, jnp.zeros((N_PAD - N_NODES, D), jnp.float32)], axis=0)

    ones_hbm = jnp.ones((CH, DEG_W), jnp.float32)
    zeros_d = jnp.zeros((ROWS_PER_SUB, DEG_W), jnp.float32)
    zeros_f = jnp.zeros((ROWS_PER_SUB, D), jnp.float32)

    deg_parts = _sc_deg(dstp, ones_hbm, zeros_d)
    dis, g = _tc_prep(deg_parts[0], deg_parts[1], x_pad)

    feat = x_pad
    h = COEFS[0] * x_pad
    for k in range(1, len(COEFS)):
        parts = _sc_round(g, srcp, dstp, zeros_f)
        feat, h, g = _tc_dense(COEFS[k], parts[0], parts[1], feat, dis, h)

    return h[:N_NODES]


# speedup vs baseline: 3.4781x; 3.4781x over previous
import functools

import jax
import jax.numpy as jnp
from jax import lax
from jax.experimental import pallas as pl
from jax.experimental.pallas import tpu as pltpu
from jax.experimental.pallas import tpu_sc as plsc

N_NODES = 10000
D = 128
E = 320000
COEFS = (1.0, -0.5, 0.25, -0.125, 0.0625)

NC = 2
NS = 16
NW = NC * NS

CH = 128
NCH = 80
NPAIR = NCH // 2
EPW = NCH * CH
E_PAD = EPW * NW
N_PAD = 10240
ROWS_PER_SUB = N_PAD // NS
DEG_W = 128


def _sc_mesh():
    return plsc.VectorSubcoreMesh(core_axis_name="c", subcore_axis_name="s")


def _sc_deg(dstp, ones_hbm, zeros_hbm):
    @functools.partial(
        pl.kernel,
        out_type=jax.ShapeDtypeStruct((NC, N_PAD, DEG_W), jnp.float32),
        mesh=_sc_mesh(),
        scratch_types=[
            pltpu.VMEM((NCH, CH), jnp.int32),
            pltpu.VMEM((CH, DEG_W), jnp.float32),
            pltpu.VMEM_SHARED((N_PAD, DEG_W), jnp.float32),
        ],
    )
    def k(dstp_hbm, ones_h, zeros_h, out_hbm, dbuf, ones_v, dega):
        c = lax.axis_index("c")
        s = lax.axis_index("s")
        wid = c * NS + s
        pltpu.sync_copy(zeros_h, dega.at[pl.ds(s * ROWS_PER_SUB, ROWS_PER_SUB)])
        pltpu.sync_copy(dstp_hbm.at[wid], dbuf)
        pltpu.sync_copy(ones_h, ones_v)
        plsc.subcore_barrier()

        def body(j, carry):
            pltpu.sync_copy(ones_v, dega.at[dbuf.at[j]], add=True)
            return carry

        lax.fori_loop(0, NCH, body, 0)
        plsc.subcore_barrier()
        sl = pl.ds(s * ROWS_PER_SUB, ROWS_PER_SUB)
        pltpu.sync_copy(dega.at[sl], out_hbm.at[c].at[sl])

    return k(dstp, ones_hbm, zeros_hbm)


def _sc_round(g, idxp, zeros_hbm):
    @functools.partial(
        pl.kernel,
        out_type=jax.ShapeDtypeStruct((NC, N_PAD, D), jnp.float32),
        mesh=_sc_mesh(),
        scratch_types=[
            pltpu.VMEM((2, CH), jnp.int32),
            pltpu.VMEM((2, CH), jnp.int32),
            pltpu.VMEM((CH, D), jnp.float32),
            pltpu.VMEM((CH, D), jnp.float32),
            pltpu.VMEM_SHARED((N_PAD, D), jnp.float32),
            pltpu.SemaphoreType.DMA,
            pltpu.SemaphoreType.DMA,
            pltpu.SemaphoreType.DMA,
            pltpu.SemaphoreType.DMA,
        ],
    )
    def k(g_hbm, idxp_hbm, zeros_h, out_hbm,
          ib0, ib1, r0, r1, agg, si0, si1, sr0, sr1):
        c = lax.axis_index("c")
        s = lax.axis_index("s")
        wid = c * NS + s
        pltpu.sync_copy(zeros_h, agg.at[pl.ds(s * ROWS_PER_SUB, ROWS_PER_SUB)])
        plsc.subcore_barrier()

        my_idx = idxp_hbm.at[wid]

        pltpu.async_copy(my_idx.at[0], ib0, si0)
        pltpu.async_copy(my_idx.at[1], ib1, si1)
        pltpu.make_async_copy(my_idx.at[0], ib0, si0).wait()
        pltpu.async_copy(g_hbm.at[ib0.at[0]], r0, sr0)
        pltpu.make_async_copy(my_idx.at[1], ib1, si1).wait()
        pltpu.async_copy(g_hbm.at[ib1.at[0]], r1, sr1)

        def body(jj, carry):
            j0 = 2 * jj
            pltpu.make_async_copy(g_hbm.at[ib0.at[0]], r0, sr0).wait()
            pltpu.sync_copy(r0, agg.at[ib0.at[1]], add=True)
            pltpu.async_copy(my_idx.at[j0 + 2], ib0, si0)
            pltpu.make_async_copy(g_hbm.at[ib1.at[0]], r1, sr1).wait()
            pltpu.sync_copy(r1, agg.at[ib1.at[1]], add=True)
            pltpu.async_copy(my_idx.at[j0 + 3], ib1, si1)
            pltpu.make_async_copy(my_idx.at[0], ib0, si0).wait()
            pltpu.async_copy(g_hbm.at[ib0.at[0]], r0, sr0)
            pltpu.make_async_copy(my_idx.at[1], ib1, si1).wait()
            pltpu.async_copy(g_hbm.at[ib1.at[0]], r1, sr1)
            return carry

        lax.fori_loop(0, NPAIR - 1, body, 0)

        pltpu.make_async_copy(g_hbm.at[ib0.at[0]], r0, sr0).wait()
        pltpu.sync_copy(r0, agg.at[ib0.at[1]], add=True)
        pltpu.make_async_copy(g_hbm.at[ib1.at[0]], r1, sr1).wait()
        pltpu.sync_copy(r1, agg.at[ib1.at[1]], add=True)

        plsc.subcore_barrier()
        sl = pl.ds(s * ROWS_PER_SUB, ROWS_PER_SUB)
        pltpu.sync_copy(agg.at[sl], out_hbm.at[c].at[sl])

    return k(g, idxp, zeros_hbm)


_TC_BLK = 1280
_TC_GRID = N_PAD // _TC_BLK


def _row_spec(w):
    return pl.BlockSpec((_TC_BLK, w), lambda i: (i, 0))


def _tc_prep(d0, d1, x):
    def body(d0_ref, d1_ref, x_ref, dis_ref, g_ref):
        deg = d0_ref[:, 0:1] + d1_ref[:, 0:1]
        dis = lax.rsqrt(jnp.maximum(deg, 1.0))
        dis_ref[...] = dis
        g_ref[...] = x_ref[...] * dis

    return pl.pallas_call(
        body,
        grid=(_TC_GRID,),
        in_specs=[_row_spec(DEG_W), _row_spec(DEG_W), _row_spec(D)],
        out_specs=[_row_spec(1), _row_spec(D)],
        out_shape=[
            jax.ShapeDtypeStruct((N_PAD, 1), jnp.float32),
            jax.ShapeDtypeStruct((N_PAD, D), jnp.float32),
        ],
    )(d0, d1, x)


def _tc_dense(theta, p0, p1, feat, dis, h):
    def body(p0_ref, p1_ref, feat_ref, dis_ref, h_ref,
             feat_o, h_o, g_o):
        agg = p0_ref[...] + p1_ref[...]
        dis = dis_ref[...]
        f = feat_ref[...] - dis * agg
        feat_o[...] = f
        h_o[...] = h_ref[...] + theta * f
        g_o[...] = f * dis

    return pl.pallas_call(
        body,
        grid=(_TC_GRID,),
        in_specs=[_row_spec(D), _row_spec(D), _row_spec(D), _row_spec(1),
                  _row_spec(D)],
        out_specs=[_row_spec(D), _row_spec(D), _row_spec(D)],
        out_shape=[
            jax.ShapeDtypeStruct((N_PAD, D), jnp.float32),
            jax.ShapeDtypeStruct((N_PAD, D), jnp.float32),
            jax.ShapeDtypeStruct((N_PAD, D), jnp.float32),
        ],
    )(p0, p1, feat, dis, h)


def kernel(x, edge_index):
    src = edge_index[0].astype(jnp.int32)
    dst = edge_index[1].astype(jnp.int32)
    pad = E_PAD - E
    srcp = jnp.concatenate([src, jnp.zeros((pad,), jnp.int32)])
    dstp = jnp.concatenate([dst, jnp.full((pad,), N_NODES, jnp.int32)])
    srcp = srcp.reshape(NW, NCH, CH)
    dstp = dstp.reshape(NW, NCH, CH)
    idxp = jnp.stack([srcp, dstp], axis=2)

    x_pad = jnp.concatenate(
        [x, jnp.zeros((N_PAD - N_NODES, D), jnp.float32)], axis=0)

    ones_hbm = jnp.ones((CH, DEG_W), jnp.float32)
    zeros_d = jnp.zeros((ROWS_PER_SUB, DEG_W), jnp.float32)
    zeros_f = jnp.zeros((ROWS_PER_SUB, D), jnp.float32)

    deg_parts = _sc_deg(dstp, ones_hbm, zeros_d)
    dis, g = _tc_prep(deg_parts[0], deg_parts[1], x_pad)

    feat = x_pad
    h = COEFS[0] * x_pad
    for k in range(1, len(COEFS)):
        parts = _sc_round(g, idxp, zeros_f)
        feat, h, g = _tc_dense(COEFS[k], parts[0], parts[1], feat, dis, h)

    return h[:N_NODES]


# speedup vs baseline: 3.5089x; 1.0088x over previous
import functools

import jax
import jax.numpy as jnp
from jax import lax
from jax.experimental import pallas as pl
from jax.experimental.pallas import tpu as pltpu
from jax.experimental.pallas import tpu_sc as plsc

N_NODES = 10000
D = 128
E = 320000
COEFS = (1.0, -0.5, 0.25, -0.125, 0.0625)

NC = 2
NS = 16
NW = NC * NS

CH = 128
NCH = 80
EPW = NCH * CH
E_PAD = EPW * NW
NCH_PAD = NCH + 4
N_PAD = 10112
ROWS_PER_SUB = N_PAD // NS
DEG_W = 128


def _sc_mesh():
    return plsc.VectorSubcoreMesh(core_axis_name="c", subcore_axis_name="s")


def _sc_deg(dstp, ones_hbm, zeros_hbm):
    @functools.partial(
        pl.kernel,
        out_type=jax.ShapeDtypeStruct((NC, N_PAD, DEG_W), jnp.float32),
        mesh=_sc_mesh(),
        scratch_types=[
            pltpu.VMEM((NCH, CH), jnp.int32),
            pltpu.VMEM((CH, DEG_W), jnp.float32),
            pltpu.VMEM_SHARED((N_PAD, DEG_W), jnp.float32),
        ],
    )
    def k(dstp_hbm, ones_h, zeros_h, out_hbm, dbuf, ones_v, dega):
        c = lax.axis_index("c")
        s = lax.axis_index("s")
        wid = c * NS + s
        pltpu.sync_copy(zeros_h, dega.at[pl.ds(s * ROWS_PER_SUB, ROWS_PER_SUB)])
        pltpu.sync_copy(dstp_hbm.at[wid], dbuf)
        pltpu.sync_copy(ones_h, ones_v)
        plsc.subcore_barrier()

        def body(j, carry):
            pltpu.sync_copy(ones_v, dega.at[dbuf.at[j]], add=True)
            return carry

        lax.fori_loop(0, NCH, body, 0)
        plsc.subcore_barrier()
        sl = pl.ds(s * ROWS_PER_SUB, ROWS_PER_SUB)
        pltpu.sync_copy(dega.at[sl], out_hbm.at[c].at[sl])

    return k(dstp, ones_hbm, zeros_hbm)


def _sc_round(g, idxp, zeros_hbm):
    @functools.partial(
        pl.kernel,
        out_type=jax.ShapeDtypeStruct((NC, N_PAD, D), jnp.float32),
        mesh=_sc_mesh(),
        scratch_types=(
            [pltpu.VMEM((2, CH), jnp.int32) for _ in range(3)]
            + [pltpu.VMEM((CH, D), jnp.float32) for _ in range(3)]
            + [pltpu.VMEM_SHARED((N_PAD, D), jnp.float32)]
            + [pltpu.SemaphoreType.DMA] * 6
        ),
    )
    def k(g_hbm, idxp_hbm, zeros_h, out_hbm, *refs):
        ib = refs[0:3]
        r = refs[3:6]
        agg = refs[6]
        si = refs[7:10]
        gr = refs[10:13]
        c = lax.axis_index("c")
        s = lax.axis_index("s")
        wid = c * NS + s
        pltpu.sync_copy(zeros_h, agg.at[pl.ds(s * ROWS_PER_SUB, ROWS_PER_SUB)])
        plsc.subcore_barrier()

        my_idx = idxp_hbm.at[wid]

        def fetch_idx(j, slot):
            pltpu.async_copy(my_idx.at[j], ib[slot], si[slot])

        def wait_idx(slot):
            pltpu.make_async_copy(my_idx.at[0], ib[slot], si[slot]).wait()

        def gather(slot):
            pltpu.async_copy(g_hbm.at[ib[slot].at[0]], r[slot], gr[slot])

        def wait_gather(slot):
            pltpu.make_async_copy(
                g_hbm.at[ib[slot].at[0]], r[slot], gr[slot]).wait()

        def scatter(slot):
            pltpu.sync_copy(r[slot], agg.at[ib[slot].at[1]], add=True)

        def step(j, b):
            b2 = (b + 2) % 3
            wait_gather(b)
            wait_idx(b2)
            gather(b2)
            scatter(b)
            fetch_idx(j + 3, b)

        fetch_idx(0, 0)
        fetch_idx(1, 1)
        wait_idx(0)
        gather(0)
        wait_idx(1)
        gather(1)
        fetch_idx(2, 2)

        def body(jg, carry):
            j = 3 * jg
            for u in range(3):
                step(j + u, u)
            return carry

        lax.fori_loop(0, 25, body, 0)

        step(75, 0)
        step(76, 1)
        wait_gather(2)
        wait_idx(1)
        gather(1)
        scatter(2)

        wait_gather((NCH - 2) % 3)
        scatter((NCH - 2) % 3)
        wait_gather((NCH - 1) % 3)
        scatter((NCH - 1) % 3)

        plsc.subcore_barrier()
        sl = pl.ds(s * ROWS_PER_SUB, ROWS_PER_SUB)
        pltpu.sync_copy(agg.at[sl], out_hbm.at[c].at[sl])

    return k(g, idxp, zeros_hbm)


_TC_BLK = 1264
_TC_GRID = N_PAD // _TC_BLK


def _row_spec(w):
    return pl.BlockSpec((_TC_BLK, w), lambda i: (i, 0))


def _tc_prep(d0, d1, x):
    def body(d0_ref, d1_ref, x_ref, dis_ref, g_ref):
        deg = d0_ref[:, 0:1] + d1_ref[:, 0:1]
        dis = lax.rsqrt(jnp.maximum(deg, 1.0))
        dis_ref[...] = dis
        g_ref[...] = x_ref[...] * dis

    return pl.pallas_call(
        body,
        grid=(_TC_GRID,),
        in_specs=[_row_spec(DEG_W), _row_spec(DEG_W), _row_spec(D)],
        out_specs=[_row_spec(1), _row_spec(D)],
        out_shape=[
            jax.ShapeDtypeStruct((N_PAD, 1), jnp.float32),
            jax.ShapeDtypeStruct((N_PAD, D), jnp.float32),
        ],
    )(d0, d1, x)


def _tc_dense(theta, p0, p1, feat, dis, h):
    def body(p0_ref, p1_ref, feat_ref, dis_ref, h_ref,
             feat_o, h_o, g_o):
        agg = p0_ref[...] + p1_ref[...]
        dis = dis_ref[...]
        f = feat_ref[...] - dis * agg
        feat_o[...] = f
        h_o[...] = h_ref[...] + theta * f
        g_o[...] = f * dis

    return pl.pallas_call(
        body,
        grid=(_TC_GRID,),
        in_specs=[_row_spec(D), _row_spec(D), _row_spec(D), _row_spec(1),
                  _row_spec(D)],
        out_specs=[_row_spec(D), _row_spec(D), _row_spec(D)],
        out_shape=[
            jax.ShapeDtypeStruct((N_PAD, D), jnp.float32),
            jax.ShapeDtypeStruct((N_PAD, D), jnp.float32),
            jax.ShapeDtypeStruct((N_PAD, D), jnp.float32),
        ],
    )(p0, p1, feat, dis, h)


def kernel(x, edge_index):
    src = edge_index[0].astype(jnp.int32)
    dst = edge_index[1].astype(jnp.int32)
    pad = E_PAD - E
    srcp = jnp.concatenate([src, jnp.zeros((pad,), jnp.int32)])
    dstp = jnp.concatenate([dst, jnp.full((pad,), N_NODES, jnp.int32)])
    srcp = srcp.reshape(NW, NCH, CH)
    dstp = dstp.reshape(NW, NCH, CH)
    idxp = jnp.stack([srcp, dstp], axis=2)
    idx_tail = jnp.tile(
        jnp.stack([jnp.zeros((CH,), jnp.int32),
                   jnp.full((CH,), N_NODES, jnp.int32)])[None, None],
        (NW, NCH_PAD - NCH, 1, 1))
    idxp = jnp.concatenate([idxp, idx_tail], axis=1)

    x_pad = jnp.concatenate(
        [x, jnp.zeros((N_PAD - N_NODES, D), jnp.float32)], axis=0)

    ones_hbm = jnp.ones((CH, DEG_W), jnp.float32)
    zeros_d = jnp.zeros((ROWS_PER_SUB, DEG_W), jnp.float32)
    zeros_f = jnp.zeros((ROWS_PER_SUB, D), jnp.float32)

    deg_parts = _sc_deg(dstp, ones_hbm, zeros_d)
    dis, g = _tc_prep(deg_parts[0], deg_parts[1], x_pad)

    feat = x_pad
    h = COEFS[0] * x_pad
    for k in range(1, len(COEFS)):
        parts = _sc_round(g, idxp, zeros_f)
        feat, h, g = _tc_dense(COEFS[k], parts[0], parts[1], feat, dis, h)

    return h[:N_NODES]
